# Initial kernel scaffold; baseline (speedup 1.0000x reference)
import jax
import jax.numpy as jnp
from jax.experimental import pallas as pl


def kernel(input_ids, table):
    raise NotImplementedError("write your pallas kernel here")


# speedup vs baseline: 1.3867x; 1.3867x over previous
import functools
import math

import jax
import jax.numpy as jnp
from jax import lax
from jax.experimental import pallas as pl
from jax.experimental.pallas import tpu as pltpu
from jax.experimental.pallas import tpu_sc as plsc

_VOCAB = 1000000
_DIM = 32
_L = 200
_B = 4096

_NC = 2
_NS = 16
_NW = _NC * _NS
_LANES = 16

_TOTAL = _B * _L
_PER_W = _TOTAL // _NW
_CS = 1600
_NCHUNK = _PER_W // _CS
_SEQ_PER_CHUNK = _CS // _L
_BATCHES = [128] * 12 + [64]
_UNROLL = 8


def _pe_tiled():
    position = jnp.arange(_L, dtype=jnp.float32)[:, None]
    div_term = jnp.exp(
        jnp.arange(0, _DIM, 2, dtype=jnp.float32) * (-math.log(10000.0) / _DIM)
    )
    ang = position * div_term
    pe = jnp.zeros((_L, _DIM), dtype=jnp.float32)
    pe = pe.at[:, 0::2].set(jnp.sin(ang))
    pe = pe.at[:, 1::2].set(jnp.cos(ang))
    return jnp.tile(pe, (_SEQ_PER_CHUNK, 1))


@functools.partial(
    pl.kernel,
    mesh=plsc.VectorSubcoreMesh(core_axis_name="c", subcore_axis_name="s"),
    out_type=jax.ShapeDtypeStruct((_TOTAL, _DIM), jnp.float32),
    scratch_types=[
        pltpu.VMEM((_CS,), jnp.int32),
        pltpu.VMEM((_CS, _DIM), jnp.float32),
        pltpu.VMEM((_CS, _DIM), jnp.float32),
        pltpu.SemaphoreType.DMA,
    ],
    compiler_params=pltpu.CompilerParams(use_tc_tiling_on_sc=False),
)
def _emb_kernel(ids_hbm, table_hbm, pe_hbm, out_hbm, idx_v, rows_v, pe_v, sem):
    wid = lax.axis_index("s") * _NC + lax.axis_index("c")
    pltpu.sync_copy(pe_hbm, pe_v)

    def run_chunk(c, carry):
        base = wid * _PER_W + c * _CS
        pltpu.sync_copy(ids_hbm.at[pl.ds(base, _CS)], idx_v)
        copies = []
        off = 0
        for bs in _BATCHES:
            copies.append(
                pltpu.async_copy(
                    table_hbm.at[idx_v.at[pl.ds(off, bs)]],
                    rows_v.at[pl.ds(off, bs)],
                    sem,
                )
            )
            off += bs
        for cp in copies:
            cp.wait()

        def add_pe(i, acc):
            r0 = i * _UNROLL
            for u in range(_UNROLL):
                r = r0 + u
                for h in range(_DIM // _LANES):
                    plsc.addupdate(
                        rows_v.at[r, pl.ds(h * _LANES, _LANES)],
                        pe_v[r, pl.ds(h * _LANES, _LANES)],
                    )
            return acc

        lax.fori_loop(0, _CS // _UNROLL, add_pe, 0)
        pltpu.sync_copy(rows_v, out_hbm.at[pl.ds(base, _CS)])
        return carry

    lax.fori_loop(0, _NCHUNK, run_chunk, 0)


def kernel(input_ids, table):
    ids_flat = input_ids.reshape(-1).astype(jnp.int32)
    out = _emb_kernel(ids_flat, table, _pe_tiled())
    return out.reshape(_B, _L, _DIM)


# speedup vs baseline: 1.4426x; 1.0403x over previous
import functools
import math

import jax
import jax.numpy as jnp
from jax import lax
from jax.experimental import pallas as pl
from jax.experimental.pallas import tpu as pltpu
from jax.experimental.pallas import tpu_sc as plsc

_VOCAB = 1000000
_DIM = 32
_L = 200
_B = 4096

_NC = 2
_NS = 16
_NW = _NC * _NS
_LANES = 16

_TOTAL = _B * _L
_PER_W = _TOTAL // _NW
_CS = 800
_NCHUNK = _PER_W // _CS
_NPAIR = _NCHUNK // 2
_SEQ_PER_CHUNK = _CS // _L
_BATCHES = [128] * 6 + [32]
_UNROLL = 8


def _pe_tiled():
    position = jnp.arange(_L, dtype=jnp.float32)[:, None]
    div_term = jnp.exp(
        jnp.arange(0, _DIM, 2, dtype=jnp.float32) * (-math.log(10000.0) / _DIM)
    )
    ang = position * div_term
    pe = jnp.zeros((_L, _DIM), dtype=jnp.float32)
    pe = pe.at[:, 0::2].set(jnp.sin(ang))
    pe = pe.at[:, 1::2].set(jnp.cos(ang))
    return jnp.tile(pe, (_SEQ_PER_CHUNK, 1))


@functools.partial(
    pl.kernel,
    mesh=plsc.VectorSubcoreMesh(core_axis_name="c", subcore_axis_name="s"),
    out_type=jax.ShapeDtypeStruct((_TOTAL, _DIM), jnp.float32),
    scratch_types=[
        pltpu.VMEM((_CS,), jnp.int32),
        pltpu.VMEM((_CS,), jnp.int32),
        pltpu.VMEM((_CS, _DIM), jnp.float32),
        pltpu.VMEM((_CS, _DIM), jnp.float32),
        pltpu.VMEM((_CS, _DIM), jnp.float32),
        pltpu.SemaphoreType.DMA,
        pltpu.SemaphoreType.DMA,
        pltpu.SemaphoreType.DMA,
        pltpu.SemaphoreType.DMA,
    ],
    compiler_params=pltpu.CompilerParams(use_tc_tiling_on_sc=False),
)
def _emb_kernel(
    ids_hbm, table_hbm, pe_hbm, out_hbm,
    idx0, idx1, rows0, rows1, pe_v,
    gsem0, gsem1, osem0, osem1,
):
    idx = (idx0, idx1)
    rows = (rows0, rows1)
    gsem = (gsem0, gsem1)
    osem = (osem0, osem1)
    wid = lax.axis_index("s") * _NC + lax.axis_index("c")
    base_w = wid * _PER_W
    pltpu.sync_copy(pe_hbm, pe_v)

    def fire(c, b):
        pltpu.sync_copy(ids_hbm.at[pl.ds(base_w + c * _CS, _CS)], idx[b])
        off = 0
        for bs in _BATCHES:
            pltpu.async_copy(
                table_hbm.at[idx[b].at[pl.ds(off, bs)]],
                rows[b].at[pl.ds(off, bs)],
                gsem[b],
            )
            off += bs

    def drain_gather(b):
        off = 0
        for bs in _BATCHES:
            pltpu.make_async_copy(
                out_hbm.at[pl.ds(off, bs)],
                rows[b].at[pl.ds(off, bs)],
                gsem[b],
            ).wait()
            off += bs

    def write_out(c, b):
        pltpu.async_copy(rows[b], out_hbm.at[pl.ds(base_w + c * _CS, _CS)], osem[b])

    def drain_write(b):
        pltpu.make_async_copy(rows[b], out_hbm.at[pl.ds(0, _CS)], osem[b]).wait()

    def add_pe(b):
        rb = rows[b]

        def body(i, acc):
            r0 = i * _UNROLL
            for u in range(_UNROLL):
                r = r0 + u
                for h in range(_DIM // _LANES):
                    plsc.addupdate(
                        rb.at[r, pl.ds(h * _LANES, _LANES)],
                        pe_v[r, pl.ds(h * _LANES, _LANES)],
                    )
            return acc

        lax.fori_loop(0, _CS // _UNROLL, body, 0)

    fire(0, 0)

    def pair(p, carry):
        c0 = 2 * p

        @pl.when(p > 0)
        def _():
            drain_write(1)

        fire(c0 + 1, 1)

        drain_gather(0)
        add_pe(0)
        write_out(c0, 0)

        drain_gather(1)
        add_pe(1)

        @pl.when(p + 1 < _NPAIR)
        def _():
            drain_write(0)
            fire(c0 + 2, 0)

        write_out(c0 + 1, 1)
        return carry

    lax.fori_loop(0, _NPAIR, pair, 0)
    drain_write(0)
    drain_write(1)


def kernel(input_ids, table):
    ids_flat = input_ids.reshape(-1).astype(jnp.int32)
    out = _emb_kernel(ids_flat, table, _pe_tiled())
    return out.reshape(_B, _L, _DIM)
